# baseline (device time: 35768 ns/iter reference)
import jax
import jax.numpy as jnp
from jax import lax
from jax.experimental import pallas as pl
from jax.experimental.pallas import tpu as pltpu

N_DEV = 16
N_TOK = 1024
D_MODEL = 256
N_EXP = 64
H = 512
CHUNK = N_TOK // N_DEV
N_PEER = N_DEV - 1
N_GRP = 4
GRP_ROWS = N_TOK // N_GRP
GRP_CHUNKS = N_DEV // N_GRP
COLW = H // 2


def kernel(x, router_W, route_idx, expert_W):
    e_local = expert_W.shape[0]

    def body(x_ref, rw_ref, idx_ref, ew_ref, out_ref,
             acc, send_bf, rs_buf,
             rs_send, rs_recv, ag_send, ag_recv):
        d = lax.axis_index("i")
        my_plane = lax.div(d, GRP_CHUNKS)

        barrier_sem = pltpu.get_barrier_semaphore()
        for k in range(N_PEER):
            q = lax.rem(d + 1 + k, N_DEV)
            pl.semaphore_signal(
                barrier_sem, inc=1,
                device_id=(q,), device_id_type=pl.DeviceIdType.MESH,
            )

        ew_bf = [ew_ref[j].astype(jnp.bfloat16) for j in range(e_local)]
        rw = rw_ref[:, :]
        ids = lax.broadcasted_iota(jnp.int32, (GRP_ROWS, N_EXP), 1)

        def rs_send_chunk(col, c, t, c_off, send_waits):
            rdma = pltpu.make_async_remote_copy(
                src_ref=send_bf.at[col, c],
                dst_ref=rs_buf.at[col, lax.rem(d - c - 1 + N_DEV, N_DEV)],
                send_sem=rs_send.at[col, t * GRP_CHUNKS + c_off],
                recv_sem=rs_recv.at[col, lax.rem(d - c - 1 + N_DEV, N_DEV)],
                device_id=(c,),
                device_id_type=pl.DeviceIdType.MESH,
            )
            if t < N_GRP - 1:
                rdma.start()
                send_waits.append((rdma, None))
            else:
                cond = c != d

                @pl.when(cond)
                def _(rdma=rdma):
                    rdma.start()

                send_waits.append((rdma, cond))

        send_waits = []
        for t in range(N_GRP):
            g = lax.rem(my_plane + 1 + t, N_GRP)
            row0 = g * GRP_ROWS
            xg = x_ref[pl.ds(row0, GRP_ROWS), :]
            scores = jnp.dot(xg, rw, preferred_element_type=jnp.float32)
            mx = jnp.max(scores, axis=1, keepdims=True)
            p = jnp.exp(scores - mx)
            p = p / jnp.sum(p, axis=1, keepdims=True)
            e0g = idx_ref[pl.ds(row0, GRP_ROWS), 0]
            e1g = idx_ref[pl.ds(row0, GRP_ROWS), 1]
            g0 = jnp.sum(jnp.where(ids == e0g[:, None], p, 0.0), axis=1)
            g1 = jnp.sum(jnp.where(ids == e1g[:, None], p, 0.0), axis=1)
            gs = g0 + g1
            w0g = g0 / gs
            w1g = g1 / gs
            pg = jnp.zeros((GRP_ROWS, H), jnp.float32)
            for j in range(e_local):
                ge = d * e_local + j
                wj = (jnp.where(e0g == ge, w0g, 0.0)
                      + jnp.where(e1g == ge, w1g, 0.0))
                xj = (xg * wj[:, None]).astype(jnp.bfloat16)
                pg = pg + jnp.dot(
                    xj, ew_bf[j], preferred_element_type=jnp.float32,
                )
            c0 = g * GRP_CHUNKS
            acc[pl.ds(c0, GRP_CHUNKS)] = pg.reshape(GRP_CHUNKS, CHUNK, H)
            pg_bf = pg.astype(jnp.bfloat16)
            send_bf[0, pl.ds(c0, GRP_CHUNKS)] = (
                pg_bf[:, :COLW].reshape(GRP_CHUNKS, CHUNK, COLW)
            )
            send_bf[1, pl.ds(c0, GRP_CHUNKS)] = (
                pg_bf[:, COLW:].reshape(GRP_CHUNKS, CHUNK, COLW)
            )
            if t == 0:
                pl.semaphore_wait(barrier_sem, N_PEER)
            for c_off in range(GRP_CHUNKS):
                rs_send_chunk(0, c0 + c_off, t, c_off, send_waits)

        for t in range(N_GRP):
            g = lax.rem(my_plane + 1 + t, N_GRP)
            for c_off in range(GRP_CHUNKS):
                rs_send_chunk(1, g * GRP_CHUNKS + c_off, t, c_off, send_waits)

        def wait_and_reduce(col):
            red = acc[d][:, col * COLW:(col + 1) * COLW]
            for k in range(N_PEER):
                recv = pltpu.make_async_remote_copy(
                    src_ref=send_bf.at[col, 0],
                    dst_ref=rs_buf.at[col, k],
                    send_sem=rs_send.at[col, 0],
                    recv_sem=rs_recv.at[col, k],
                    device_id=(d,),
                    device_id_type=pl.DeviceIdType.MESH,
                )
                recv.wait_recv()
                red = red + rs_buf[col, k].astype(jnp.float32)
            out_ref[pl.ds(d * CHUNK, CHUNK), pl.ds(col * COLW, COLW)] = (
                red.astype(jnp.bfloat16)
            )

        def ag_broadcast(col):
            rdmas = []
            for k in range(N_PEER):
                q = lax.rem(d + 1 + k, N_DEV)
                slot = N_PEER - 1 - k
                rdma = pltpu.make_async_remote_copy(
                    src_ref=out_ref.at[pl.ds(d * CHUNK, CHUNK),
                                       pl.ds(col * COLW, COLW)],
                    dst_ref=out_ref.at[pl.ds(d * CHUNK, CHUNK),
                                       pl.ds(col * COLW, COLW)],
                    send_sem=ag_send.at[col, slot],
                    recv_sem=ag_recv.at[col, slot],
                    device_id=(q,),
                    device_id_type=pl.DeviceIdType.MESH,
                )
                rdma.start()
                rdmas.append(rdma)
            return rdmas

        wait_and_reduce(0)
        ag0 = ag_broadcast(0)
        wait_and_reduce(1)
        ag1 = ag_broadcast(1)

        for rdma, cond in send_waits:
            if cond is None:
                rdma.wait_send()
            else:
                @pl.when(cond)
                def _(rdma=rdma):
                    rdma.wait_send()

        for rdma in ag0 + ag1:
            rdma.wait()

    return pl.pallas_call(
        body,
        out_shape=jax.ShapeDtypeStruct((N_TOK, H), jnp.bfloat16),
        in_specs=[pl.BlockSpec(memory_space=pltpu.VMEM)] * 4,
        out_specs=pl.BlockSpec(memory_space=pltpu.VMEM),
        scratch_shapes=[
            pltpu.VMEM((N_DEV, CHUNK, H), jnp.float32),
            pltpu.VMEM((2, N_DEV, CHUNK, COLW), jnp.bfloat16),
            pltpu.VMEM((2, N_DEV, CHUNK, COLW), jnp.bfloat16),
            pltpu.SemaphoreType.DMA((2, N_DEV)),
            pltpu.SemaphoreType.DMA((2, N_DEV)),
            pltpu.SemaphoreType.DMA((2, N_DEV)),
            pltpu.SemaphoreType.DMA((2, N_DEV)),
        ],
        compiler_params=pltpu.CompilerParams(collective_id=0),
    )(x, router_W, route_idx, expert_W)
